# baseline (speedup 1.0000x reference)
import functools

import jax
import jax.numpy as jnp
from jax.experimental import pallas as pl
from jax.experimental.pallas import tpu as pltpu


def _se_gate_kernel(x_ref, w1_ref, w2_ref, y_ref, *, inv_hw):
    pooled = jnp.sum(x_ref[...], axis=-1) * inv_hw
    h = jnp.maximum(
        jnp.dot(pooled, w1_ref[...], preferred_element_type=jnp.float32), 0.0)
    y = jax.nn.sigmoid(
        jnp.dot(h, w2_ref[...], preferred_element_type=jnp.float32))
    y_ref[...] = y.astype(y_ref.dtype)


def kernel(x, w1_t, w2_t):
    B, C, H, W = x.shape
    HW = H * W
    Cr = w1_t.shape[1]
    xr = x.reshape(B, C, HW)

    BB = 8
    while B % BB != 0:
        BB //= 2

    gates = pl.pallas_call(
        functools.partial(_se_gate_kernel, inv_hw=1.0 / HW),
        out_shape=jax.ShapeDtypeStruct((B, C), jnp.float32),
        grid_spec=pltpu.PrefetchScalarGridSpec(
            num_scalar_prefetch=0,
            grid=(B // BB,),
            in_specs=[
                pl.BlockSpec((BB, C, HW), lambda b: (b, 0, 0)),
                pl.BlockSpec((C, Cr), lambda b: (0, 0)),
                pl.BlockSpec((Cr, C), lambda b: (0, 0)),
            ],
            out_specs=pl.BlockSpec((BB, C), lambda b: (b, 0)),
        ),
        compiler_params=pltpu.CompilerParams(
            dimension_semantics=("parallel",),
            vmem_limit_bytes=64 * 1024 * 1024,
        ),
    )(xr, w1_t, w2_t)

    return x * gates[:, :, None, None]
